# Initial kernel scaffold; baseline (speedup 1.0000x reference)
import jax
import jax.numpy as jnp
from jax.experimental import pallas as pl


def kernel(edges_supervised, h_first, h_second):
    raise NotImplementedError("write your pallas kernel here")


# speedup vs baseline: 1.5345x; 1.5345x over previous
import functools

import jax
import jax.numpy as jnp
from jax import lax
from jax.experimental import pallas as pl
from jax.experimental.pallas import tpu as pltpu
from jax.experimental.pallas import tpu_sc as plsc

N_NODES = 10000
N_EDGES = 320000
D_FEAT = 128

NC = 2
NS = 16
L = 16
NW = NC * NS
EPW = N_EDGES // NW
CHUNK = 400
NGROUP = CHUNK // L
NCHUNK = EPW // CHUNK


def _normalize_body(hf_ref, hs_ref, of_ref, os_ref):
    hf = hf_ref[...]
    hs = hs_ref[...]
    of_ref[...] = hf * lax.rsqrt(jnp.sum(hf * hf, axis=1, keepdims=True))
    os_ref[...] = hs * lax.rsqrt(jnp.sum(hs * hs, axis=1, keepdims=True))


def _normalize(h_first, h_second):
    rows = h_first.shape[0]
    blk = 2000
    grid = rows // blk
    spec = pl.BlockSpec((blk, D_FEAT), lambda i: (i, 0))
    return pl.pallas_call(
        _normalize_body,
        grid=(grid,),
        in_specs=[spec, spec],
        out_specs=[spec, spec],
        out_shape=[
            jax.ShapeDtypeStruct(h_first.shape, jnp.float32),
            jax.ShapeDtypeStruct(h_second.shape, jnp.float32),
        ],
    )(h_first, h_second)


def _sc_body(idx0_hbm, idx1_hbm, hf_hbm, hs_hbm, out_hbm,
             i0_v, i1_v, rows_a, rows_b, out_v, sem_a, sem_b):
    wid = lax.axis_index("s") * NC + lax.axis_index("c")
    base = wid * EPW

    def chunk_body(c, carry):
        start = base + c * CHUNK
        pltpu.sync_copy(idx0_hbm.at[pl.ds(start, CHUNK)], i0_v)
        pltpu.sync_copy(idx1_hbm.at[pl.ds(start, CHUNK)], i1_v)
        cp_a = pltpu.async_copy(hf_hbm.at[i0_v], rows_a, sem_a)
        cp_b = pltpu.async_copy(hs_hbm.at[i1_v], rows_b, sem_b)
        cp_a.wait()
        cp_b.wait()

        lanes = lax.iota(jnp.int32, L)

        def group_body(g, carry2):
            rows = g * L + lanes
            acc = jnp.zeros((L,), jnp.float32)

            def dot_body(j, acc):
                col = jnp.full((L,), j, jnp.int32)
                a = plsc.load_gather(rows_a, [rows, col])
                b = plsc.load_gather(rows_b, [rows, col])
                return acc + a * b

            acc = lax.fori_loop(0, D_FEAT, dot_body, acc)
            out_v[pl.ds(g * L, L)] = acc
            return carry2

        lax.fori_loop(0, NGROUP, group_body, 0)
        pltpu.sync_copy(out_v, out_hbm.at[pl.ds(start, CHUNK)])
        return carry

    lax.fori_loop(0, NCHUNK, chunk_body, 0)


@functools.partial(jax.jit, static_argnames=())
def _sc_edge_dots(idx0, idx1, hf_n, hs_n):
    mesh = plsc.VectorSubcoreMesh(core_axis_name="c", subcore_axis_name="s")
    return pl.kernel(
        _sc_body,
        out_type=jax.ShapeDtypeStruct((N_EDGES,), jnp.float32),
        mesh=mesh,
        compiler_params=pltpu.CompilerParams(needs_layout_passes=False),
        scratch_types=[
            pltpu.VMEM((CHUNK,), jnp.int32),
            pltpu.VMEM((CHUNK,), jnp.int32),
            pltpu.VMEM((CHUNK, D_FEAT), jnp.float32),
            pltpu.VMEM((CHUNK, D_FEAT), jnp.float32),
            pltpu.VMEM((CHUNK,), jnp.float32),
            pltpu.SemaphoreType.DMA,
            pltpu.SemaphoreType.DMA,
        ],
    )(idx0, idx1, hf_n, hs_n)


def kernel(edges_supervised, h_first, h_second):
    idx0 = edges_supervised[0].astype(jnp.int32)
    idx1 = edges_supervised[1].astype(jnp.int32)
    hf_n, hs_n = _normalize(h_first, h_second)
    return _sc_edge_dots(idx0, idx1, hf_n, hs_n)
